# baseline (speedup 1.0000x reference)
import functools

import jax
import jax.numpy as jnp
from jax import lax
from jax.experimental import pallas as pl
from jax.experimental.pallas import tpu as pltpu
from jax.experimental.pallas import tpu_sc as plsc

MAX_LEN = 4096
BATCH = 2
D_MODEL = 1024

NUM_CORES = 2
NUM_SUBCORES = 16
LANES = 16
NUM_WORKERS = NUM_CORES * NUM_SUBCORES

P_PER_WORKER = MAX_LEN // NUM_WORKERS
CHUNK_P = 16
N_CHUNKS = P_PER_WORKER // CHUNK_P

_mesh = plsc.VectorSubcoreMesh(core_axis_name="c", subcore_axis_name="s")


@functools.partial(
    pl.kernel,
    mesh=_mesh,
    out_type=jax.ShapeDtypeStruct((MAX_LEN, BATCH, D_MODEL), jnp.float32),
    scratch_types=[
        pltpu.VMEM((CHUNK_P, BATCH, D_MODEL), jnp.float32),
        pltpu.VMEM((CHUNK_P, D_MODEL), jnp.float32),
    ],
)
def _sc_add(x_hbm, e_hbm, o_hbm, xb, eb):
    wid = lax.axis_index("s") * NUM_CORES + lax.axis_index("c")
    base_p = wid * P_PER_WORKER

    def chunk_body(k, carry):
        p0 = base_p + k * CHUNK_P
        pltpu.sync_copy(x_hbm.at[pl.ds(p0, CHUNK_P)], xb)
        pltpu.sync_copy(e_hbm.at[pl.ds(p0, CHUNK_P)], eb)

        def row_body(i, c):
            for j in range(D_MODEL // LANES):
                sl = pl.ds(j * LANES, LANES)
                ev = eb[i, sl]
                xb[i, 0, sl] = xb[i, 0, sl] + ev
                xb[i, 1, sl] = xb[i, 1, sl] + ev
            return c

        lax.fori_loop(0, CHUNK_P, row_body, 0)
        pltpu.sync_copy(xb, o_hbm.at[pl.ds(p0, CHUNK_P)])
        return carry

    lax.fori_loop(0, N_CHUNKS, chunk_body, 0)


def kernel(x, emb_table):
    return _sc_add(x, emb_table)


# speedup vs baseline: 3.6429x; 3.6429x over previous
import jax
import jax.numpy as jnp
from jax.experimental import pallas as pl
from jax.experimental.pallas import tpu as pltpu

MAX_LEN = 4096
BATCH = 2
D_MODEL = 1024

BLOCK_P = 512


def _add_body(x_ref, e_ref, o_ref):
    e = e_ref[...]
    for b in range(BATCH):
        o_ref[:, b, :] = x_ref[:, b, :] + e


def kernel(x, emb_table):
    grid = (MAX_LEN // BLOCK_P,)
    return pl.pallas_call(
        _add_body,
        grid=grid,
        in_specs=[
            pl.BlockSpec((BLOCK_P, BATCH, D_MODEL), lambda i: (i, 0, 0)),
            pl.BlockSpec((BLOCK_P, D_MODEL), lambda i: (i, 0)),
        ],
        out_specs=pl.BlockSpec((BLOCK_P, BATCH, D_MODEL), lambda i: (i, 0, 0)),
        out_shape=jax.ShapeDtypeStruct((MAX_LEN, BATCH, D_MODEL), jnp.float32),
        compiler_params=pltpu.CompilerParams(
            dimension_semantics=("parallel",),
        ),
    )(x, emb_table)
